# baseline (speedup 1.0000x reference)
import jax
import jax.numpy as jnp
from jax import lax
from jax.experimental import pallas as pl
from jax.experimental.pallas import tpu as pltpu
from jax.experimental.pallas import tpu_sc as plsc

MAX_LEN = 8192
DIM = 1024
NUM_WORKERS = 32
ROWS_PER_WORKER = MAX_LEN // NUM_WORKERS
CHUNK = 32
CHUNKS_PER_WORKER = ROWS_PER_WORKER // CHUNK

_mesh = plsc.VectorSubcoreMesh(core_axis_name="c", subcore_axis_name="s")


def _sc_body(t_hbm, pe_hbm, zeros_hbm, out_hbm, t_v):
    wid = lax.axis_index("s") * 2 + lax.axis_index("c")
    base = wid * ROWS_PER_WORKER

    pltpu.sync_copy(t_hbm, t_v)
    t = t_v[...][0]

    @pl.when(base + ROWS_PER_WORKER <= t)
    def _copy_all():
        pltpu.sync_copy(pe_hbm.at[pl.ds(base, ROWS_PER_WORKER)],
                        out_hbm.at[pl.ds(base, ROWS_PER_WORKER)])

    @pl.when(base >= t)
    def _zero_all():
        pltpu.sync_copy(zeros_hbm,
                        out_hbm.at[pl.ds(base, ROWS_PER_WORKER)])

    @pl.when(jnp.logical_and(base < t, base + ROWS_PER_WORKER > t))
    def _straddle():
        for ci in range(CHUNKS_PER_WORKER):
            cbase = base + ci * CHUNK

            @pl.when(cbase + CHUNK <= t)
            def _copy_chunk():
                pltpu.sync_copy(pe_hbm.at[pl.ds(cbase, CHUNK)],
                                out_hbm.at[pl.ds(cbase, CHUNK)])

            @pl.when(cbase >= t)
            def _zero_chunk():
                pltpu.sync_copy(zeros_hbm.at[pl.ds(0, CHUNK)],
                                out_hbm.at[pl.ds(cbase, CHUNK)])

            @pl.when(jnp.logical_and(cbase < t, cbase + CHUNK > t))
            def _row_chunk():
                def row_body(r, carry):
                    @pl.when(cbase + r < t)
                    def _copy_row():
                        pltpu.sync_copy(pe_hbm.at[cbase + r],
                                        out_hbm.at[cbase + r])

                    @pl.when(cbase + r >= t)
                    def _zero_row():
                        pltpu.sync_copy(zeros_hbm.at[0],
                                        out_hbm.at[cbase + r])

                    return carry

                lax.fori_loop(0, CHUNK, row_body, 0)


_sc_call = pl.kernel(
    _sc_body,
    mesh=_mesh,
    out_type=jax.ShapeDtypeStruct((MAX_LEN, DIM), jnp.float32),
    scratch_types=[pltpu.VMEM((16,), jnp.int32)],
)


def kernel(pe, T):
    t_arr = jnp.full((16,), T, dtype=jnp.int32)
    zeros = jnp.zeros((ROWS_PER_WORKER, DIM), dtype=jnp.float32)
    out = _sc_call(t_arr, pe, zeros)
    return out[None, :, :]


# speedup vs baseline: 22.5191x; 22.5191x over previous
import jax
import jax.numpy as jnp
from jax import lax
from jax.experimental import pallas as pl
from jax.experimental.pallas import tpu as pltpu
from jax.experimental.pallas import tpu_sc as plsc

MAX_LEN = 8192
DIM = 1024
NUM_WORKERS = 32
ROWS_PER_WORKER = MAX_LEN // NUM_WORKERS
CHUNK = 32
CHUNKS_PER_WORKER = ROWS_PER_WORKER // CHUNK

_mesh = plsc.VectorSubcoreMesh(core_axis_name="c", subcore_axis_name="s")


def _sc_body(t_hbm, pe_hbm, zeros_hbm, out_hbm,
             t_v, buf0, buf1, si0, si1, so0, so1):
    wid = lax.axis_index("s") * 2 + lax.axis_index("c")
    base = wid * ROWS_PER_WORKER

    pltpu.sync_copy(t_hbm, t_v)
    t = t_v[...][0]

    bufs = (buf0, buf1)
    sin = (si0, si1)
    sout = (so0, so1)

    @pl.when(base + ROWS_PER_WORKER <= t)
    def _fast_copy():
        h_in = [None] * CHUNKS_PER_WORKER
        h_out = [None] * CHUNKS_PER_WORKER
        for i in range(CHUNKS_PER_WORKER):
            b = i & 1
            if i >= 2:
                h_out[i - 2].wait()
            src = pe_hbm.at[pl.ds(base + i * CHUNK, CHUNK)]
            h_in[i] = pltpu.async_copy(src, bufs[b], sin[b])
            h_in[i].wait()
            dst = out_hbm.at[pl.ds(base + i * CHUNK, CHUNK)]
            h_out[i] = pltpu.async_copy(bufs[b], dst, sout[b])
        h_out[CHUNKS_PER_WORKER - 2].wait()
        h_out[CHUNKS_PER_WORKER - 1].wait()

    @pl.when(base + ROWS_PER_WORKER > t)
    def _masked_path():
        for ci in range(CHUNKS_PER_WORKER):
            cbase = base + ci * CHUNK

            @pl.when(cbase + CHUNK <= t)
            def _copy_chunk():
                pltpu.sync_copy(pe_hbm.at[pl.ds(cbase, CHUNK)], buf0)
                pltpu.sync_copy(buf0, out_hbm.at[pl.ds(cbase, CHUNK)])

            @pl.when(cbase >= t)
            def _zero_chunk():
                pltpu.sync_copy(zeros_hbm.at[pl.ds(0, CHUNK)], buf0)
                pltpu.sync_copy(buf0, out_hbm.at[pl.ds(cbase, CHUNK)])

            @pl.when(jnp.logical_and(cbase < t, cbase + CHUNK > t))
            def _straddle_chunk():
                pltpu.sync_copy(pe_hbm.at[pl.ds(cbase, CHUNK)], buf0)

                def row_body(r, carry):
                    @pl.when(cbase + r >= t)
                    def _zero_row():
                        pltpu.sync_copy(zeros_hbm.at[0], buf0.at[r])

                    return carry

                lax.fori_loop(0, CHUNK, row_body, 0)
                pltpu.sync_copy(buf0, out_hbm.at[pl.ds(cbase, CHUNK)])


_sc_call = pl.kernel(
    _sc_body,
    mesh=_mesh,
    out_type=jax.ShapeDtypeStruct((MAX_LEN, DIM), jnp.float32),
    scratch_types=[
        pltpu.VMEM((16,), jnp.int32),
        pltpu.VMEM((CHUNK, DIM), jnp.float32),
        pltpu.VMEM((CHUNK, DIM), jnp.float32),
        pltpu.SemaphoreType.DMA,
        pltpu.SemaphoreType.DMA,
        pltpu.SemaphoreType.DMA,
        pltpu.SemaphoreType.DMA,
    ],
)


def kernel(pe, T):
    t_arr = jnp.full((16,), T, dtype=jnp.int32)
    zeros = jnp.zeros((CHUNK, DIM), dtype=jnp.float32)
    out = _sc_call(t_arr, pe, zeros)
    return out[None, :, :]
